# Initial kernel scaffold; baseline (speedup 1.0000x reference)
import jax
import jax.numpy as jnp
from jax.experimental import pallas as pl


def kernel(x, edge_index, batch, MD, params):
    raise NotImplementedError("write your pallas kernel here")


# speedup vs baseline: 2.5301x; 2.5301x over previous
import functools

import jax
import jax.numpy as jnp
import numpy as np
from jax import lax
from jax.experimental import pallas as pl
from jax.experimental.pallas import tpu as pltpu
from jax.experimental.pallas import tpu_sc as plsc

N = 10000
E = 320000
D = 128
H = 128
B = 64
C = 10

NC = 2
NS = 16
K = 128
CH = 158
E_PAD = NS * CH * K
SPLIT = N // 2
ACC_ROWS = 5120
RPS = ACC_ROWS // NS

_HIGHEST = jax.lax.Precision.HIGHEST


def _sc_scatter_body(h_hbm, srcw_hbm, dst0w_hbm, dst1w_hbm, zeros_hbm,
                     out0_hbm, out1_hbm,
                     src_v, dst_v, rows0, rows1, zv, acc, sem0, sem1):
    c = lax.axis_index("c")
    s = lax.axis_index("s")

    pltpu.sync_copy(srcw_hbm.at[s], src_v)

    @pl.when(c == 0)
    def _():
        pltpu.sync_copy(dst0w_hbm.at[s], dst_v)

    @pl.when(c == 1)
    def _():
        pltpu.sync_copy(dst1w_hbm.at[s], dst_v)

    pltpu.sync_copy(zeros_hbm, zv)
    base = s * RPS
    pltpu.sync_copy(zv, acc.at[pl.ds(base, 128)])
    pltpu.sync_copy(zv, acc.at[pl.ds(base + 128, 128)])
    pltpu.sync_copy(zv.at[pl.ds(0, 64)], acc.at[pl.ds(base + 256, 64)])
    plsc.subcore_barrier()

    pltpu.async_copy(h_hbm.at[src_v.at[0]], rows0, sem0)
    pltpu.async_copy(h_hbm.at[src_v.at[1]], rows1, sem1)

    @pl.loop(0, CH, step=2)
    def _(j):
        pltpu.make_async_copy(h_hbm.at[src_v.at[j]], rows0, sem0).wait()
        pltpu.sync_copy(rows0, acc.at[dst_v.at[j]], add=True)

        @pl.when(j + 2 < CH)
        def _():
            pltpu.async_copy(h_hbm.at[src_v.at[j + 2]], rows0, sem0)

        pltpu.make_async_copy(h_hbm.at[src_v.at[j + 1]], rows1, sem1).wait()
        pltpu.sync_copy(rows1, acc.at[dst_v.at[j + 1]], add=True)

        @pl.when(j + 3 < CH)
        def _():
            pltpu.async_copy(h_hbm.at[src_v.at[j + 3]], rows1, sem1)

    plsc.subcore_barrier()

    @pl.when(c == 0)
    def _():
        pltpu.sync_copy(acc.at[pl.ds(base, RPS)], out0_hbm.at[pl.ds(base, RPS)])

    @pl.when(c == 1)
    def _():
        pltpu.sync_copy(acc.at[pl.ds(base, RPS)], out1_hbm.at[pl.ds(base, RPS)])


@functools.lru_cache(maxsize=1)
def _make_sc_scatter():
    return pl.kernel(
        _sc_scatter_body,
        out_type=(jax.ShapeDtypeStruct((ACC_ROWS, H), jnp.float32),
                  jax.ShapeDtypeStruct((ACC_ROWS, H), jnp.float32)),
        mesh=plsc.VectorSubcoreMesh(core_axis_name="c", subcore_axis_name="s",
                                    num_cores=NC, num_subcores=NS),
        scratch_types=[
            pltpu.VMEM((CH, K), jnp.int32),
            pltpu.VMEM((CH, K), jnp.int32),
            pltpu.VMEM((K, H), jnp.float32),
            pltpu.VMEM((K, H), jnp.float32),
            pltpu.VMEM((K, H), jnp.float32),
            pltpu.VMEM_SHARED((ACC_ROWS, H), jnp.float32),
            pltpu.SemaphoreType.DMA,
            pltpu.SemaphoreType.DMA,
        ],
    )


def _sc_scatter(h, srcw, dst0w, dst1w, zeros_blk):
    return _make_sc_scatter()(h, srcw, dst0w, dst1w, zeros_blk)


TILE_R = 5000


def _mlp_body(x_ref, a0_ref, a1_ref, er_ref, w1_ref, b1_ref, w2_ref, b2_ref,
              o_ref):
    i = pl.program_id(0)
    a = jnp.where(i == 0, a0_ref[...], a1_ref[...])
    h = x_ref[...] * er_ref[...] + a
    h = jnp.dot(h, w1_ref[...], preferred_element_type=jnp.float32,
                precision=_HIGHEST) + b1_ref[...]
    h = jnp.maximum(h, 0.0)
    h = jnp.dot(h, w2_ref[...], preferred_element_type=jnp.float32,
                precision=_HIGHEST) + b2_ref[...]
    o_ref[...] = jnp.maximum(h, 0.0)


def _gin_mlp(x, a0, a1, epsrow, w1, b1, w2, b2):
    grid = (N // TILE_R,)
    row_spec = pl.BlockSpec((TILE_R, H), lambda i: (i, 0))
    a0_spec = pl.BlockSpec((TILE_R, H), lambda i: (0, 0))
    a1_spec = pl.BlockSpec((TILE_R, H), lambda i: (0, 0))
    full = pl.BlockSpec((1, H), lambda i: (0, 0))
    wspec = pl.BlockSpec((H, H), lambda i: (0, 0))
    return pl.pallas_call(
        _mlp_body,
        grid=grid,
        in_specs=[row_spec, a0_spec, a1_spec, full, wspec, full, wspec, full],
        out_specs=row_spec,
        out_shape=jax.ShapeDtypeStruct((N, H), jnp.float32),
    )(x, a0, a1, epsrow, w1, b1, w2, b2)


def _bn_body(h_ref, g_ref, b_ref, o_ref):
    h = h_ref[...]
    mu = jnp.mean(h, axis=0, keepdims=True)
    d = h - mu
    var = jnp.mean(d * d, axis=0, keepdims=True)
    o_ref[...] = d * lax.rsqrt(var + 1e-5) * g_ref[...] + b_ref[...]


def _bn(h, gamma, beta):
    return pl.pallas_call(
        _bn_body,
        out_shape=jax.ShapeDtypeStruct((N, H), jnp.float32),
    )(h, gamma, beta)


def _head_body(h_ref, batch_ref, mdp_ref, k1_ref, b1_ref, k2_ref, b2_ref,
               fcw_ref, fcb_ref, l1w_ref, l1b_ref, l2wa_ref, l2wb_ref,
               l2b_ref, l3w_ref, l3b_ref, o_ref):
    y = jnp.dot(mdp_ref[...], k1_ref[...], preferred_element_type=jnp.float32,
                precision=_HIGHEST) + b1_ref[...]
    m = jnp.maximum(jnp.maximum(y[:, 0:256], y[:, 256:512]),
                    jnp.maximum(y[:, 512:768], y[:, 768:1024]))
    p = jnp.maximum(m[:, 0:216], 0.0)
    xm = jnp.dot(p, k2_ref[...], preferred_element_type=jnp.float32,
                 precision=_HIGHEST) + b2_ref[...]
    xm = jnp.maximum(xm, 0.0)
    xm = jnp.dot(xm, fcw_ref[...], preferred_element_type=jnp.float32,
                 precision=_HIGHEST) + fcb_ref[...]
    xm = jnp.maximum(xm, 0.0)

    onehot = (batch_ref[...] ==
              lax.broadcasted_iota(jnp.int32, (B, N), 0)).astype(jnp.float32)
    sums = jnp.dot(onehot, h_ref[...], preferred_element_type=jnp.float32,
                   precision=_HIGHEST)
    cnt = jnp.sum(onehot, axis=1, keepdims=True)
    g = sums / jnp.maximum(cnt, 1.0)

    g = jnp.dot(g, l1w_ref[...], preferred_element_type=jnp.float32,
                precision=_HIGHEST) + l1b_ref[...]
    g = jnp.maximum(g, 0.0)
    z = (jnp.dot(g, l2wa_ref[...], preferred_element_type=jnp.float32,
                 precision=_HIGHEST)
         + jnp.dot(xm, l2wb_ref[...], preferred_element_type=jnp.float32,
                   precision=_HIGHEST) + l2b_ref[...])
    z = jnp.maximum(z, 0.0)
    z = jnp.dot(z, l3w_ref[...], preferred_element_type=jnp.float32,
                precision=_HIGHEST) + l3b_ref[...]

    zmax = jnp.max(z, axis=1, keepdims=True)
    ez = jnp.exp(z - zmax)
    lse = zmax + jnp.log(jnp.sum(ez, axis=1, keepdims=True))
    o_ref[...] = z - lse


def _head(h3, batch_row, mdp, k1, b1, k2, b2, fcw, fcb, l1w, l1b, l2wa, l2wb,
          l2b, l3w, l3b):
    return pl.pallas_call(
        _head_body,
        out_shape=jax.ShapeDtypeStruct((B, C), jnp.float32),
    )(h3, batch_row, mdp, k1, b1, k2, b2, fcw, fcb, l1w, l1b, l2wa, l2wb,
      l2b, l3w, l3b)


def _conv1_maps():
    rows, cols, taps = [], [], []
    bcols, bo = [], []
    for a_i in (0, 1):
        for a_j in (0, 1):
            a = a_i * 2 + a_j
            for o in range(6):
                for i2 in range(6):
                    for j2 in range(6):
                        col = a * 256 + o * 36 + i2 * 6 + j2
                        bcols.append(col)
                        bo.append(o)
                        for di in range(5):
                            for dj in range(5):
                                p_ = (2 * i2 + a_i + di) * 16 + (2 * j2 + a_j + dj)
                                rows.append(p_)
                                cols.append(col)
                                taps.append(o * 25 + di * 5 + dj)
    return (np.array(rows), np.array(cols), np.array(taps),
            np.array(bcols), np.array(bo))


def _conv2_maps():
    rows, cols, taps = [], [], []
    for o in range(16):
        for i in (0, 1):
            for j in (0, 1):
                col = o * 4 + i * 2 + j
                for cch in range(6):
                    for di in range(5):
                        for dj in range(5):
                            q = cch * 36 + (i + di) * 6 + (j + dj)
                            rows.append(q)
                            cols.append(col)
                            taps.append(((o * 6 + cch) * 5 + di) * 5 + dj)
    return np.array(rows), np.array(cols), np.array(taps)


_C1R, _C1C, _C1T, _C1BC, _C1BO = _conv1_maps()
_C2R, _C2C, _C2T = _conv2_maps()


def kernel(x, edge_index, batch, MD, params):
    src = edge_index[0].astype(jnp.int32)
    dst = edge_index[1].astype(jnp.int32)
    pad = E_PAD - E
    srcw = jnp.concatenate([src, jnp.zeros((pad,), jnp.int32)]).reshape(NS, CH, K)
    dstp = jnp.concatenate([dst, jnp.full((pad,), N, jnp.int32)])
    dst0 = jnp.where(dstp < SPLIT, dstp, SPLIT).reshape(NS, CH, K)
    dst1 = jnp.where(dstp >= SPLIT, dstp - SPLIT, SPLIT).reshape(NS, CH, K)
    zeros_blk = jnp.zeros((K, H), jnp.float32)

    batch_row = batch.astype(jnp.int32).reshape(1, N)

    p = params
    w1f = p['cw1'].reshape(-1)
    k1 = jnp.zeros((256, 1024), jnp.float32).at[_C1R, _C1C].add(w1f[_C1T])
    b1 = jnp.zeros((1, 1024), jnp.float32).at[0, _C1BC].set(p['cb1'][_C1BO])
    w2f = p['cw2'].reshape(-1)
    k2 = jnp.zeros((216, 64), jnp.float32).at[_C2R, _C2C].add(w2f[_C2T])
    b2 = jnp.tile(p['cb2'][:, None], (1, 4)).reshape(1, 64)
    mdp = jnp.pad(MD.reshape(B, 12, 12), ((0, 0), (2, 2), (2, 2))).reshape(B, 256)

    h = x
    for li in range(3):
        gp = p['gin%d' % li]
        a0, a1 = _sc_scatter(h, srcw, dst0, dst1, zeros_blk)
        epsrow = (1.0 + gp['eps']) * jnp.ones((1, H), jnp.float32)
        h = _gin_mlp(h, a0, a1, epsrow,
                     gp['W1'], gp['b1'].reshape(1, H),
                     gp['W2'], gp['b2'].reshape(1, H))
        h = _bn(h, gp['gamma'].reshape(1, H), gp['beta'].reshape(1, H))

    return _head(h, batch_row, mdp, k1, b1, k2, b2,
                 p['fcW'], p['fcb'].reshape(1, 64),
                 p['l1W'], p['l1b'].reshape(1, H),
                 p['l2W'][:H], p['l2W'][H:], p['l2b'].reshape(1, 64),
                 p['l3W'], p['l3b'].reshape(1, C))
